# Initial kernel scaffold; baseline (speedup 1.0000x reference)
import jax
import jax.numpy as jnp
from jax.experimental import pallas as pl


def kernel(hidden_states, gate_weight):
    raise NotImplementedError("write your pallas kernel here")


# speedup vs baseline: 1.0822x; 1.0822x over previous
import functools

import jax
import jax.numpy as jnp
from jax.experimental import pallas as pl
from jax.experimental.pallas import tpu as pltpu

TOP_K = 8
NUM_EXPERTS = 64
HIDDEN = 768
TOKENS = 32768
BT = 1024


def _router_block(hs_ref, gw_ref, w_ref, i_ref):
    logits = jax.lax.dot_general(
        hs_ref[...], gw_ref[...],
        dimension_numbers=(((1,), (1,)), ((), ())),
        preferred_element_type=jnp.float32,
    )
    col = jax.lax.broadcasted_iota(jnp.int32, (BT, NUM_EXPERTS), 1)
    x = logits
    neg_inf = jnp.float32(-jnp.inf)
    vals = []
    idxs = []
    for _ in range(TOP_K):
        m = jnp.max(x, axis=1, keepdims=True)
        hit = x >= m
        a = jnp.min(jnp.where(hit, col, NUM_EXPERTS), axis=1,
                    keepdims=True)
        vals.append(m)
        idxs.append(a)
        x = jnp.where(col == a, neg_inf, x)
    v = jnp.concatenate(vals, axis=1)
    e = jnp.exp(v - v[:, 0:1])
    w_ref[...] = e / jnp.sum(e, axis=1, keepdims=True)
    i_ref[...] = jnp.concatenate(idxs, axis=1)


@jax.jit
def kernel(hidden_states, gate_weight):
    grid = (TOKENS // BT,)
    w, i = pl.pallas_call(
        _router_block,
        grid=grid,
        in_specs=[
            pl.BlockSpec((BT, HIDDEN), lambda t: (t, 0)),
            pl.BlockSpec((NUM_EXPERTS, HIDDEN), lambda t: (0, 0)),
        ],
        out_specs=[
            pl.BlockSpec((BT, TOP_K), lambda t: (t, 0)),
            pl.BlockSpec((BT, TOP_K), lambda t: (t, 0)),
        ],
        out_shape=[
            jax.ShapeDtypeStruct((TOKENS, TOP_K), jnp.float32),
            jax.ShapeDtypeStruct((TOKENS, TOP_K), jnp.int32),
        ],
        compiler_params=pltpu.CompilerParams(
            dimension_semantics=("arbitrary",),
        ),
    )(hidden_states, gate_weight)
    return (w, i)


# speedup vs baseline: 2.3506x; 2.1721x over previous
import jax
import jax.numpy as jnp
from jax.experimental import pallas as pl
from jax.experimental.pallas import tpu as pltpu

TOP_K = 8
NUM_EXPERTS = 64
HIDDEN = 768
TOKENS = 32768
BT = 512


def _router_block(hs_ref, gw_ref, w_ref, i_ref):
    lt = jax.lax.dot_general(
        gw_ref[...], hs_ref[...],
        dimension_numbers=(((1,), (1,)), ((), ())),
        preferred_element_type=jnp.float32,
    )
    row = jax.lax.broadcasted_iota(jnp.int32, (NUM_EXPERTS, BT), 0)
    x = lt
    neg_inf = jnp.float32(-jnp.inf)
    vals = []
    idxs = []
    for _ in range(TOP_K):
        m = jnp.max(x, axis=0, keepdims=True)
        hit = x >= m
        a = jnp.min(jnp.where(hit, row, NUM_EXPERTS), axis=0,
                    keepdims=True)
        vals.append(m)
        idxs.append(a)
        x = jnp.where(row == a, neg_inf, x)
    v = jnp.concatenate(vals, axis=0)
    e = jnp.exp(v - v[0:1, :])
    w_ref[...] = e / jnp.sum(e, axis=0, keepdims=True)
    i_ref[...] = jnp.concatenate(idxs, axis=0)


@jax.jit
def kernel(hidden_states, gate_weight):
    grid = (TOKENS // BT,)
    w, i = pl.pallas_call(
        _router_block,
        grid=grid,
        in_specs=[
            pl.BlockSpec((BT, HIDDEN), lambda t: (t, 0)),
            pl.BlockSpec((NUM_EXPERTS, HIDDEN), lambda t: (0, 0)),
        ],
        out_specs=[
            pl.BlockSpec((TOP_K, BT), lambda t: (0, t)),
            pl.BlockSpec((TOP_K, BT), lambda t: (0, t)),
        ],
        out_shape=[
            jax.ShapeDtypeStruct((TOP_K, TOKENS), jnp.float32),
            jax.ShapeDtypeStruct((TOP_K, TOKENS), jnp.int32),
        ],
        compiler_params=pltpu.CompilerParams(
            dimension_semantics=("arbitrary",),
        ),
    )(hidden_states, gate_weight)
    return (w.T, i.T)


# speedup vs baseline: 3.2322x; 1.3751x over previous
import jax
import jax.numpy as jnp
from jax.experimental import pallas as pl
from jax.experimental.pallas import tpu as pltpu

TOP_K = 8
NUM_EXPERTS = 64
HIDDEN = 768
TOKENS = 32768
BT = 1024


def _router_block(hs_ref, gw_ref, w_ref, i_ref):
    lt = jax.lax.dot_general(
        gw_ref[...], hs_ref[...],
        dimension_numbers=(((1,), (1,)), ((), ())),
        preferred_element_type=jnp.float32,
    )
    row = jax.lax.broadcasted_iota(jnp.int32, (NUM_EXPERTS, BT), 0)
    x = lt
    neg_inf = jnp.float32(-jnp.inf)
    vals = []
    idxs = []
    for _ in range(TOP_K):
        m = jnp.max(x, axis=0, keepdims=True)
        hit = x >= m
        a = jnp.min(jnp.where(hit, row, NUM_EXPERTS), axis=0,
                    keepdims=True)
        vals.append(m)
        idxs.append(a)
        x = jnp.where(row == a, neg_inf, x)
    v = jnp.concatenate(vals, axis=0)
    e = jnp.exp(v - v[0:1, :])
    w_ref[...] = e / jnp.sum(e, axis=0, keepdims=True)
    i_ref[...] = jnp.concatenate(idxs, axis=0)


@jax.jit
def kernel(hidden_states, gate_weight):
    grid = (TOKENS // BT,)
    w, i = pl.pallas_call(
        _router_block,
        grid=grid,
        in_specs=[
            pl.BlockSpec((BT, HIDDEN), lambda t: (t, 0)),
            pl.BlockSpec((NUM_EXPERTS, HIDDEN), lambda t: (0, 0)),
        ],
        out_specs=[
            pl.BlockSpec((TOP_K, BT), lambda t: (0, t)),
            pl.BlockSpec((TOP_K, BT), lambda t: (0, t)),
        ],
        out_shape=[
            jax.ShapeDtypeStruct((TOP_K, TOKENS), jnp.float32),
            jax.ShapeDtypeStruct((TOP_K, TOKENS), jnp.int32),
        ],
        compiler_params=pltpu.CompilerParams(
            dimension_semantics=("arbitrary",),
        ),
    )(hidden_states, gate_weight)
    return (w.T, i.T)


# speedup vs baseline: 3.9657x; 1.2269x over previous
import jax
import jax.numpy as jnp
from jax.experimental import pallas as pl
from jax.experimental.pallas import tpu as pltpu

TOP_K = 8
NUM_EXPERTS = 64
HIDDEN = 768
TOKENS = 32768
BT = 2048


def _router_block(hs_ref, gw_ref, w_ref, i_ref):
    lt = jax.lax.dot_general(
        gw_ref[...], hs_ref[...],
        dimension_numbers=(((1,), (1,)), ((), ())),
        preferred_element_type=jnp.float32,
    )
    row = jax.lax.broadcasted_iota(jnp.int32, (NUM_EXPERTS, BT), 0)
    x = lt
    neg_inf = jnp.float32(-jnp.inf)
    vals = []
    idxs = []
    for _ in range(TOP_K):
        m = jnp.max(x, axis=0, keepdims=True)
        hit = x >= m
        a = jnp.min(jnp.where(hit, row, NUM_EXPERTS), axis=0,
                    keepdims=True)
        vals.append(m)
        idxs.append(a)
        x = jnp.where(row == a, neg_inf, x)
    v = jnp.concatenate(vals, axis=0)
    e = jnp.exp(v - v[0:1, :])
    w_ref[...] = e / jnp.sum(e, axis=0, keepdims=True)
    i_ref[...] = jnp.concatenate(idxs, axis=0)


@jax.jit
def kernel(hidden_states, gate_weight):
    grid = (TOKENS // BT,)
    w, i = pl.pallas_call(
        _router_block,
        grid=grid,
        in_specs=[
            pl.BlockSpec((BT, HIDDEN), lambda t: (t, 0)),
            pl.BlockSpec((NUM_EXPERTS, HIDDEN), lambda t: (0, 0)),
        ],
        out_specs=[
            pl.BlockSpec((TOP_K, BT), lambda t: (0, t)),
            pl.BlockSpec((TOP_K, BT), lambda t: (0, t)),
        ],
        out_shape=[
            jax.ShapeDtypeStruct((TOP_K, TOKENS), jnp.float32),
            jax.ShapeDtypeStruct((TOP_K, TOKENS), jnp.int32),
        ],
        compiler_params=pltpu.CompilerParams(
            dimension_semantics=("arbitrary",),
        ),
    )(hidden_states, gate_weight)
    return (w.T, i.T)


# speedup vs baseline: 4.3628x; 1.1001x over previous
import jax
import jax.numpy as jnp
from jax.experimental import pallas as pl
from jax.experimental.pallas import tpu as pltpu

TOP_K = 8
NUM_EXPERTS = 64
HIDDEN = 768
TOKENS = 32768
BT = 4096


def _router_block(hs_ref, gw_ref, w_ref, i_ref):
    lt = jax.lax.dot_general(
        gw_ref[...], hs_ref[...],
        dimension_numbers=(((1,), (1,)), ((), ())),
        preferred_element_type=jnp.float32,
    )
    row = jax.lax.broadcasted_iota(jnp.int32, (NUM_EXPERTS, BT), 0)
    x = lt
    neg_inf = jnp.float32(-jnp.inf)
    vals = []
    idxs = []
    for _ in range(TOP_K):
        m = jnp.max(x, axis=0, keepdims=True)
        hit = x >= m
        a = jnp.min(jnp.where(hit, row, NUM_EXPERTS), axis=0,
                    keepdims=True)
        vals.append(m)
        idxs.append(a)
        x = jnp.where(row == a, neg_inf, x)
    v = jnp.concatenate(vals, axis=0)
    e = jnp.exp(v - v[0:1, :])
    w_ref[...] = e / jnp.sum(e, axis=0, keepdims=True)
    i_ref[...] = jnp.concatenate(idxs, axis=0)


@jax.jit
def kernel(hidden_states, gate_weight):
    grid = (TOKENS // BT,)
    w, i = pl.pallas_call(
        _router_block,
        grid=grid,
        in_specs=[
            pl.BlockSpec((BT, HIDDEN), lambda t: (t, 0)),
            pl.BlockSpec((NUM_EXPERTS, HIDDEN), lambda t: (0, 0)),
        ],
        out_specs=[
            pl.BlockSpec((TOP_K, BT), lambda t: (0, t)),
            pl.BlockSpec((TOP_K, BT), lambda t: (0, t)),
        ],
        out_shape=[
            jax.ShapeDtypeStruct((TOP_K, TOKENS), jnp.float32),
            jax.ShapeDtypeStruct((TOP_K, TOKENS), jnp.int32),
        ],
        compiler_params=pltpu.CompilerParams(
            dimension_semantics=("arbitrary",),
        ),
    )(hidden_states, gate_weight)
    return (w.T, i.T)
